# baseline (speedup 1.0000x reference)
import functools

import jax
import jax.numpy as jnp
from jax import lax
from jax.experimental import pallas as pl
from jax.experimental.pallas import tpu as pltpu
from jax.experimental.pallas import tpu_sc as plsc

D = 64
B = 16384
NC = 2
NS = 16
NW = NC * NS
BPW = B // NW
CHUNK = 128
NCH = BPW // CHUNK

BM = 2048

NT = 3
NITEMS = NT * NCH
KSLOTS = 6
GFLIGHT = 5

V0, V2, V4, V5 = 16, 400, 4, 64


def _pad_body(t0, t1, t2, o0, o1, o2):
    z = jnp.zeros_like(t0[...])
    o0[...] = jnp.concatenate([t0[...], z], axis=1)
    o1[...] = jnp.concatenate([t1[...], z], axis=1)
    o2[...] = jnp.concatenate([t2[...], z], axis=1)


def _pad_tables(tables):
    v = tables[0].shape[0]
    nb = 10
    rb = v // nb
    in_spec = pl.BlockSpec((rb, D), lambda i: (i, 0))
    out_spec = pl.BlockSpec((rb, 2 * D), lambda i: (i, 0))
    return pl.pallas_call(
        _pad_body,
        grid=(nb,),
        in_specs=[in_spec] * 3,
        out_specs=[out_spec] * 3,
        out_shape=[jax.ShapeDtypeStruct((v, 2 * D), jnp.float32)] * 3,
    )(*tables)


def _sc_gather3(idx_list, tables):
    mesh = plsc.VectorSubcoreMesh(core_axis_name="c", subcore_axis_name="s")

    @functools.partial(
        pl.kernel,
        mesh=mesh,
        compiler_params=pltpu.CompilerParams(use_tc_tiling_on_sc=True),
        out_type=[jax.ShapeDtypeStruct((B, 2 * D), jnp.float32)] * NT,
        scratch_types=[
            pltpu.VMEM((NT, NCH, CHUNK), jnp.int32),
            pltpu.VMEM((KSLOTS, CHUNK, 2 * D), jnp.float32),
            pltpu.SemaphoreType.DMA((NT,)),
            pltpu.SemaphoreType.DMA((KSLOTS,)),
            pltpu.SemaphoreType.DMA((KSLOTS,)),
        ],
    )
    def k(*refs):
        idx_refs = refs[0:NT]
        tab_refs = refs[NT:2 * NT]
        out_refs = refs[2 * NT:3 * NT]
        idx_v, rows_v, isem, gsem, wsem = refs[3 * NT:]
        wid = lax.axis_index("s") * NC + lax.axis_index("c")
        base = wid * BPW

        icopies = [
            pltpu.async_copy(idx_refs[t].at[wid], idx_v.at[t], isem.at[t])
            for t in range(NT)
        ]
        idx_ready = set()

        def gather(i):
            t, j = divmod(i, NCH)
            s = i % KSLOTS
            if t not in idx_ready:
                icopies[t].wait()
                idx_ready.add(t)
            return pltpu.async_copy(
                tab_refs[t].at[idx_v.at[t, j]], rows_v.at[s], gsem.at[s])

        def writeout(i):
            t, j = divmod(i, NCH)
            s = i % KSLOTS
            return pltpu.async_copy(
                rows_v.at[s],
                out_refs[t].at[pl.ds(base + j * CHUNK, CHUNK)],
                wsem.at[s])

        gcopies = [None] * NITEMS
        wcopies = [None] * NITEMS
        for i in range(GFLIGHT):
            gcopies[i] = gather(i)
        for i in range(NITEMS):
            gcopies[i].wait()
            wcopies[i] = writeout(i)
            n = i + GFLIGHT
            if n < NITEMS:
                prev = n - KSLOTS
                if prev >= 0:
                    wcopies[prev].wait()
                gcopies[n] = gather(n)
        for i in range(NITEMS - KSLOTS, NITEMS):
            wcopies[i].wait()

    return k(*idx_list, *tables)


def _pre_body(e0, e2, e4, e5, w0, w2, w4, w5, w7, wn, bn, bfin,
              m0, m2, m4, m5, w7n, c0):
    m0[...] = jnp.dot(e0[...], w0[...], preferred_element_type=jnp.float32)
    m2[...] = jnp.dot(e2[...], w2[...], preferred_element_type=jnp.float32)
    m4[...] = jnp.dot(e4[...], w4[...], preferred_element_type=jnp.float32)
    m5[...] = jnp.dot(e5[...], w5[...], preferred_element_type=jnp.float32)
    w7n[...] = jnp.dot(wn[...], w7[...], preferred_element_type=jnp.float32)
    c0[...] = (jnp.dot(bn[...], w7[...], preferred_element_type=jnp.float32)
               + bfin[...])


def _precompute(e0, e2, e4, e5, W_final, W_num, bn2d, bf2d):
    wslice = lambda t: pl.BlockSpec((D, D), lambda i, t=t: (t, 0))
    full = lambda v: pl.BlockSpec((v, D), lambda i: (0, 0))
    one = pl.BlockSpec((1, D), lambda i: (0, 0))
    return pl.pallas_call(
        _pre_body,
        grid=(1,),
        in_specs=[full(V0), full(V2), full(V4), full(V5),
                  wslice(0), wslice(2), wslice(4), wslice(5), wslice(7),
                  one, one, one],
        out_specs=[full(V0), full(V2), full(V4), full(V5), one, one],
        out_shape=[
            jax.ShapeDtypeStruct((V0, D), jnp.float32),
            jax.ShapeDtypeStruct((V2, D), jnp.float32),
            jax.ShapeDtypeStruct((V4, D), jnp.float32),
            jax.ShapeDtypeStruct((V5, D), jnp.float32),
            jax.ShapeDtypeStruct((1, D), jnp.float32),
            jax.ShapeDtypeStruct((1, D), jnp.float32),
        ],
    )(e0, e2, e4, e5, W_final, W_final, W_final, W_final, W_final,
      W_num, bn2d, bf2d)


def _onehot_dot(idx2d, v, m):
    cols = lax.broadcasted_iota(jnp.int32, (BM, v), 1)
    oh = (cols == idx2d).astype(jnp.float32)
    return jnp.dot(oh, m, preferred_element_type=jnp.float32)


def _tc_body(f1, f3, f6, i0, i2, i4, i5, freq, w1, w3, w6,
             m0, m2, m4, m5, w7n, c0, out):
    acc = c0[...] + freq[...] * w7n[...]
    w1s = jnp.concatenate([w1[...], w1[...]], axis=0)
    w3s = jnp.concatenate([w3[...], w3[...]], axis=0)
    w6s = jnp.concatenate([w6[...], w6[...]], axis=0)
    acc = acc + jnp.dot(f1[...], w1s, preferred_element_type=jnp.float32)
    acc = acc + jnp.dot(f3[...], w3s, preferred_element_type=jnp.float32)
    acc = acc + jnp.dot(f6[...], w6s, preferred_element_type=jnp.float32)
    acc = acc + _onehot_dot(i0[...], V0, m0[...])
    acc = acc + _onehot_dot(i2[...], V2, m2[...])
    acc = acc + _onehot_dot(i4[...], V4, m4[...])
    acc = acc + _onehot_dot(i5[...], V5, m5[...])
    out[...] = acc


def _tc_project(f1, f3, f6, idx_small, freq2d, W_final, pre):
    m0, m2, m4, m5, w7n, c0 = pre
    feat_spec = pl.BlockSpec((BM, 2 * D), lambda i: (i, 0))
    row_spec = pl.BlockSpec((BM, D), lambda i: (i, 0))
    col1 = pl.BlockSpec((BM, 1), lambda i: (i, 0))
    wslice = lambda t: pl.BlockSpec((D, D), lambda i, t=t: (t, 0))
    full = lambda v: pl.BlockSpec((v, D), lambda i: (0, 0))
    one = pl.BlockSpec((1, D), lambda i: (0, 0))
    return pl.pallas_call(
        _tc_body,
        grid=(B // BM,),
        in_specs=[feat_spec, feat_spec, feat_spec,
                  col1, col1, col1, col1, col1,
                  wslice(1), wslice(3), wslice(6),
                  full(V0), full(V2), full(V4), full(V5), one, one],
        out_specs=row_spec,
        out_shape=jax.ShapeDtypeStruct((B, D), jnp.float32),
    )(f1, f3, f6, *idx_small, freq2d, W_final, W_final, W_final,
      m0, m2, m4, m5, w7n, c0)


def kernel(base_mut, b_id, amino_mut, a_id, amino_flag, protein_region, c_id,
           freq_value,
           emb_base_mut, emb_b_id, emb_amino_mut, emb_a_id, emb_amino_flag,
           emb_protein_region, emb_c_id,
           W_num, b_num, W_final, b_final):
    idx_list = [a.reshape(NW, NCH, CHUNK) for a in (b_id, a_id, c_id)]
    tables = _pad_tables([emb_b_id, emb_a_id, emb_c_id])

    feats = _sc_gather3(idx_list, tables)

    bn2d = b_num.reshape(1, D)
    bf2d = b_final.reshape(1, D)
    pre = _precompute(emb_base_mut, emb_amino_mut, emb_amino_flag,
                      emb_protein_region, W_final, W_num, bn2d, bf2d)
    idx_small = [a.reshape(B, 1)
                 for a in (base_mut, amino_mut, amino_flag, protein_region)]
    freq2d = freq_value.reshape(B, 1)
    return _tc_project(feats[0], feats[1], feats[2], idx_small, freq2d,
                       W_final, pre)


# speedup vs baseline: 1.2025x; 1.2025x over previous
import functools

import jax
import jax.numpy as jnp
from jax import lax
from jax.experimental import pallas as pl
from jax.experimental.pallas import tpu as pltpu
from jax.experimental.pallas import tpu_sc as plsc

D = 64
B = 16384
NC = 2
NS = 16
NW = NC * NS
BPW = B // NW
CHUNK = 128
NCH = BPW // CHUNK

BM = 2048

NT = 3
NITEMS = NT * NCH
KSLOTS = 7
GFLIGHT = 6

V0, V2, V4, V5 = 16, 400, 4, 64


def _sc_gather3(idx_list, tables):
    mesh = plsc.VectorSubcoreMesh(core_axis_name="c", subcore_axis_name="s")

    @functools.partial(
        pl.kernel,
        mesh=mesh,
        compiler_params=pltpu.CompilerParams(use_tc_tiling_on_sc=True),
        out_type=[jax.ShapeDtypeStruct((B, 2 * D), jnp.float32)] * NT,
        scratch_types=[
            pltpu.VMEM((NT, NCH, CHUNK), jnp.int32),
            pltpu.VMEM((KSLOTS, CHUNK, 2 * D), jnp.float32),
            pltpu.SemaphoreType.DMA((NT,)),
            pltpu.SemaphoreType.DMA((KSLOTS,)),
            pltpu.SemaphoreType.DMA((KSLOTS,)),
        ],
    )
    def k(*refs):
        idx_refs = refs[0:NT]
        tab_refs = refs[NT:2 * NT]
        out_refs = refs[2 * NT:3 * NT]
        idx_v, rows_v, isem, gsem, wsem = refs[3 * NT:]
        wid = lax.axis_index("s") * NC + lax.axis_index("c")
        base = wid * BPW

        icopies = [
            pltpu.async_copy(idx_refs[t].at[wid], idx_v.at[t], isem.at[t])
            for t in range(NT)
        ]
        idx_ready = set()

        def gather(i):
            t, j = divmod(i, NCH)
            s = i % KSLOTS
            if t not in idx_ready:
                icopies[t].wait()
                idx_ready.add(t)
            return pltpu.async_copy(
                tab_refs[t].at[idx_v.at[t, j]], rows_v.at[s], gsem.at[s])

        def writeout(i):
            t, j = divmod(i, NCH)
            s = i % KSLOTS
            return pltpu.async_copy(
                rows_v.at[s],
                out_refs[t].at[pl.ds(base + j * CHUNK, CHUNK)],
                wsem.at[s])

        gcopies = [None] * NITEMS
        wcopies = [None] * NITEMS
        for i in range(GFLIGHT):
            gcopies[i] = gather(i)
        for i in range(NITEMS):
            gcopies[i].wait()
            wcopies[i] = writeout(i)
            n = i + GFLIGHT
            if n < NITEMS:
                prev = n - KSLOTS
                if prev >= 0:
                    wcopies[prev].wait()
                gcopies[n] = gather(n)
        for i in range(NITEMS - KSLOTS, NITEMS):
            wcopies[i].wait()

    return k(*idx_list, *tables)


def _pre_body(e0, e2, e4, e5, w0, w2, w4, w5, w7, wn, bn, bfin,
              m0, m2, m4, m5, w7n, c0):
    m0[...] = jnp.dot(e0[...], w0[...], preferred_element_type=jnp.float32)
    m2[...] = jnp.dot(e2[...], w2[...], preferred_element_type=jnp.float32)
    m4[...] = jnp.dot(e4[...], w4[...], preferred_element_type=jnp.float32)
    m5[...] = jnp.dot(e5[...], w5[...], preferred_element_type=jnp.float32)
    w7n[...] = jnp.dot(wn[...], w7[...], preferred_element_type=jnp.float32)
    c0[...] = (jnp.dot(bn[...], w7[...], preferred_element_type=jnp.float32)
               + bfin[...])


def _precompute(e0, e2, e4, e5, W_final, W_num, bn2d, bf2d):
    wslice = lambda t: pl.BlockSpec((D, D), lambda i, t=t: (t, 0))
    full = lambda v: pl.BlockSpec((v, D), lambda i: (0, 0))
    one = pl.BlockSpec((1, D), lambda i: (0, 0))
    return pl.pallas_call(
        _pre_body,
        grid=(1,),
        in_specs=[full(V0), full(V2), full(V4), full(V5),
                  wslice(0), wslice(2), wslice(4), wslice(5), wslice(7),
                  one, one, one],
        out_specs=[full(V0), full(V2), full(V4), full(V5), one, one],
        out_shape=[
            jax.ShapeDtypeStruct((V0, D), jnp.float32),
            jax.ShapeDtypeStruct((V2, D), jnp.float32),
            jax.ShapeDtypeStruct((V4, D), jnp.float32),
            jax.ShapeDtypeStruct((V5, D), jnp.float32),
            jax.ShapeDtypeStruct((1, D), jnp.float32),
            jax.ShapeDtypeStruct((1, D), jnp.float32),
        ],
    )(e0, e2, e4, e5, W_final, W_final, W_final, W_final, W_final,
      W_num, bn2d, bf2d)


def _onehot_dot(idx2d, v, m):
    cols = lax.broadcasted_iota(jnp.int32, (BM, v), 1)
    oh = (cols == idx2d).astype(jnp.float32)
    return jnp.dot(oh, m, preferred_element_type=jnp.float32)


def _tc_body(f1, f3, f6, i0, i2, i4, i5, freq, w1, w3, w6,
             m0, m2, m4, m5, w7n, c0, out):
    acc = c0[...] + freq[...] * w7n[...]
    w1s = jnp.concatenate([w1[...], w1[...]], axis=0)
    w3s = jnp.concatenate([w3[...], w3[...]], axis=0)
    w6s = jnp.concatenate([w6[...], w6[...]], axis=0)
    acc = acc + jnp.dot(f1[...], w1s, preferred_element_type=jnp.float32)
    acc = acc + jnp.dot(f3[...], w3s, preferred_element_type=jnp.float32)
    acc = acc + jnp.dot(f6[...], w6s, preferred_element_type=jnp.float32)
    acc = acc + _onehot_dot(i0[...], V0, m0[...])
    acc = acc + _onehot_dot(i2[...], V2, m2[...])
    acc = acc + _onehot_dot(i4[...], V4, m4[...])
    acc = acc + _onehot_dot(i5[...], V5, m5[...])
    out[...] = acc


def _tc_project(f1, f3, f6, idx_small, freq2d, W_final, pre):
    m0, m2, m4, m5, w7n, c0 = pre
    feat_spec = pl.BlockSpec((BM, 2 * D), lambda i: (i, 0))
    row_spec = pl.BlockSpec((BM, D), lambda i: (i, 0))
    col1 = pl.BlockSpec((BM, 1), lambda i: (i, 0))
    wslice = lambda t: pl.BlockSpec((D, D), lambda i, t=t: (t, 0))
    full = lambda v: pl.BlockSpec((v, D), lambda i: (0, 0))
    one = pl.BlockSpec((1, D), lambda i: (0, 0))
    return pl.pallas_call(
        _tc_body,
        grid=(B // BM,),
        in_specs=[feat_spec, feat_spec, feat_spec,
                  col1, col1, col1, col1, col1,
                  wslice(1), wslice(3), wslice(6),
                  full(V0), full(V2), full(V4), full(V5), one, one],
        out_specs=row_spec,
        out_shape=jax.ShapeDtypeStruct((B, D), jnp.float32),
    )(f1, f3, f6, *idx_small, freq2d, W_final, W_final, W_final,
      m0, m2, m4, m5, w7n, c0)


def kernel(base_mut, b_id, amino_mut, a_id, amino_flag, protein_region, c_id,
           freq_value,
           emb_base_mut, emb_b_id, emb_amino_mut, emb_a_id, emb_amino_flag,
           emb_protein_region, emb_c_id,
           W_num, b_num, W_final, b_final):
    idx_list = [a.reshape(NW, NCH, CHUNK) for a in (b_id, a_id, c_id)]
    tables = [
        jnp.concatenate([t, jnp.zeros(t.shape, t.dtype)], axis=1)
        for t in (emb_b_id, emb_a_id, emb_c_id)
    ]

    feats = _sc_gather3(idx_list, tables)

    bn2d = b_num.reshape(1, D)
    bf2d = b_final.reshape(1, D)
    pre = _precompute(emb_base_mut, emb_amino_mut, emb_amino_flag,
                      emb_protein_region, W_final, W_num, bn2d, bf2d)
    idx_small = [a.reshape(B, 1)
                 for a in (base_mut, amino_mut, amino_flag, protein_region)]
    freq2d = freq_value.reshape(B, 1)
    return _tc_project(feats[0], feats[1], feats[2], idx_small, freq2d,
                       W_final, pre)
